# Initial kernel scaffold; baseline (speedup 1.0000x reference)
import jax
import jax.numpy as jnp
from jax.experimental import pallas as pl


def kernel(x, edge_index, Wp, bp, Wc, bc, gamma, beta):
    raise NotImplementedError("write your pallas kernel here")


# speedup vs baseline: 10.7271x; 10.7271x over previous
import functools

import jax
import jax.numpy as jnp
from jax import lax
from jax.experimental import pallas as pl
from jax.experimental.pallas import tpu as pltpu
from jax.experimental.pallas import tpu_sc as plsc

N = 10000
E = 320000
D = 128
L = 3

NC = 2
NS = 16
NW = NC * NS
C = 128
NCH = 79
PE = NW * NCH * C
NPAD = 10240
RPT = NPAD // NS


_MESH = plsc.VectorSubcoreMesh(core_axis_name="c", subcore_axis_name="s")


def _zero_vmem_rows(buf, nrows, ncols):
    z = jnp.zeros((16,), jnp.float32)

    def row(i, _):
        for j in range(ncols // 16):
            buf[i, pl.ds(j * 16, 16)] = z
        return 0

    lax.fori_loop(0, nrows, row, 0)


@functools.partial(
    pl.kernel,
    out_type=jax.ShapeDtypeStruct((NC, NPAD, D), jnp.float32),
    mesh=_MESH,
    scratch_types=[
        pltpu.VMEM((NCH, C), jnp.int32),
        pltpu.VMEM((C, D), jnp.float32),
        pltpu.VMEM_SHARED((NPAD, D), jnp.float32),
    ],
)
def _sc_degree(dst_hbm, out_hbm, dst_v, ones_v, deg_sh):
    c = lax.axis_index("c")
    s = lax.axis_index("s")
    w = c * NS + s
    pltpu.sync_copy(dst_hbm.at[w], dst_v)
    _zero_vmem_rows(ones_v, C, D)
    for r in range(RPT // C):
        pltpu.sync_copy(ones_v, deg_sh.at[pl.ds(s * RPT + r * C, C)])
    one = jnp.ones((16,), jnp.float32)

    def fill(i, _):
        for j in range(D // 16):
            ones_v[i, pl.ds(j * 16, 16)] = one
        return 0

    lax.fori_loop(0, C, fill, 0)
    plsc.subcore_barrier()

    def chunk(g, _):
        pltpu.sync_copy(ones_v, deg_sh.at[dst_v.at[g]], add=True)
        return 0

    lax.fori_loop(0, NCH, chunk, 0)
    plsc.subcore_barrier()
    pltpu.sync_copy(deg_sh.at[pl.ds(s * RPT, RPT)],
                    out_hbm.at[c, pl.ds(s * RPT, RPT)])


@functools.partial(
    pl.kernel,
    out_type=jax.ShapeDtypeStruct((NC, NPAD, D), jnp.float32),
    mesh=_MESH,
    scratch_types=[
        pltpu.VMEM((NCH, C), jnp.int32),
        pltpu.VMEM((NCH, C), jnp.int32),
        pltpu.VMEM((C, D), jnp.float32),
        pltpu.VMEM_SHARED((NPAD, D), jnp.float32),
        pltpu.SemaphoreType.DMA,
    ],
)
def _sc_scatter(y_hbm, src_hbm, dst_hbm, out_hbm, src_v, dst_v, rows_v,
                acc_sh, sem):
    c = lax.axis_index("c")
    s = lax.axis_index("s")
    w = c * NS + s
    pltpu.sync_copy(src_hbm.at[w], src_v)
    pltpu.sync_copy(dst_hbm.at[w], dst_v)
    _zero_vmem_rows(rows_v, C, D)
    for r in range(RPT // C):
        pltpu.sync_copy(rows_v, acc_sh.at[pl.ds(s * RPT + r * C, C)])
    plsc.subcore_barrier()

    def chunk(g, _):
        pltpu.async_copy(y_hbm.at[src_v.at[g]], rows_v, sem).wait()
        pltpu.sync_copy(rows_v, acc_sh.at[dst_v.at[g]], add=True)
        return 0

    lax.fori_loop(0, NCH, chunk, 0)
    plsc.subcore_barrier()
    pltpu.sync_copy(acc_sh.at[pl.ds(s * RPT, RPT)],
                    out_hbm.at[c, pl.ds(s * RPT, RPT)])


BM = 1024
GRID = (N + BM - 1) // BM


def _dinv_of(degp_ref):
    deg = degp_ref[0, :, 0:1] + degp_ref[1, :, 0:1] + 1.0
    return lax.rsqrt(deg)


def _pre_body(x_ref, wp_ref, bp_ref, wc0_ref, degp_ref, h_ref, y_ref):
    h = jnp.dot(x_ref[...], wp_ref[...],
                preferred_element_type=jnp.float32) + bp_ref[...]
    h_ref[...] = h
    y_ref[...] = jnp.dot(h * _dinv_of(degp_ref), wc0_ref[...],
                         preferred_element_type=jnp.float32)


def _make_layer_body(residual, has_next):
    def body(parts_ref, y_ref, hprev_ref, degp_ref, bc_ref, g_ref, b_ref,
             *rest):
        if has_next:
            wc_ref, h_ref, ynext_ref = rest
        else:
            wc_ref = None
            (h_ref,) = rest
        dinv = _dinv_of(degp_ref)
        acc = parts_ref[0] + parts_ref[1] + y_ref[...]
        t = acc * dinv + bc_ref[...]
        mu = jnp.mean(t, axis=-1, keepdims=True)
        tc = t - mu
        var = jnp.mean(tc * tc, axis=-1, keepdims=True)
        t = tc * lax.rsqrt(var + 1e-5) * g_ref[...] + b_ref[...]
        t = jnp.maximum(t, 0.0)
        if residual:
            t = t + hprev_ref[...]
        h_ref[...] = t
        if has_next:
            ynext_ref[...] = jnp.dot(t * dinv, wc_ref[...],
                                     preferred_element_type=jnp.float32)

    return body


def _row_spec():
    return pl.BlockSpec((BM, D), lambda i: (i, 0))


def _full_spec(shape):
    nd = len(shape)
    return pl.BlockSpec(shape, lambda i: (0,) * nd)


def _degp_spec():
    return pl.BlockSpec((2, BM, D), lambda i: (0, i, 0))


def _tc_pre(x, Wp, bp, Wc0, degp):
    return pl.pallas_call(
        _pre_body,
        grid=(GRID,),
        in_specs=[
            _row_spec(),
            _full_spec((D, D)),
            _full_spec((1, D)),
            _full_spec((D, D)),
            _degp_spec(),
        ],
        out_specs=[_row_spec(), _row_spec()],
        out_shape=[
            jax.ShapeDtypeStruct((N, D), jnp.float32),
            jax.ShapeDtypeStruct((N, D), jnp.float32),
        ],
    )(x, Wp, bp.reshape(1, D), Wc0, degp)


def _tc_layer(parts, y, hprev, degp, bc_i, g_i, b_i, wc_next, residual):
    has_next = wc_next is not None
    ins = [parts, y, hprev, degp, bc_i.reshape(1, D), g_i.reshape(1, D),
           b_i.reshape(1, D)]
    in_specs = [
        pl.BlockSpec((2, BM, D), lambda i: (0, i, 0)),
        _row_spec(),
        _row_spec(),
        _degp_spec(),
        _full_spec((1, D)),
        _full_spec((1, D)),
        _full_spec((1, D)),
    ]
    if has_next:
        ins.append(wc_next)
        in_specs.append(_full_spec((D, D)))
        out_specs = [_row_spec(), _row_spec()]
        out_shape = [
            jax.ShapeDtypeStruct((N, D), jnp.float32),
            jax.ShapeDtypeStruct((N, D), jnp.float32),
        ]
    else:
        out_specs = [_row_spec()]
        out_shape = [jax.ShapeDtypeStruct((N, D), jnp.float32)]
    res = pl.pallas_call(
        _make_layer_body(residual, has_next),
        grid=(GRID,),
        in_specs=in_specs,
        out_specs=out_specs,
        out_shape=out_shape,
    )(*ins)
    return res if has_next else (res[0], None)


def kernel(x, edge_index, Wp, bp, Wc, bc, gamma, beta):
    src = edge_index[0]
    dst = edge_index[1]
    pad = PE - E
    srcp = jnp.concatenate([src, jnp.zeros((pad,), jnp.int32)])
    dstp = jnp.concatenate([dst, jnp.full((pad,), N, jnp.int32)])
    srcp = srcp.reshape(NW, NCH, C)
    dstp = dstp.reshape(NW, NCH, C)

    degp = _sc_degree(dstp)
    h, y = _tc_pre(x, Wp, bp, Wc[0], degp)
    for i in range(L):
        parts = _sc_scatter(y, srcp, dstp)
        wc_next = Wc[i + 1] if i < L - 1 else None
        h, y = _tc_layer(parts, y, h, degp, bc[i], gamma[i], beta[i],
                         wc_next, residual=(i > 0))
    return h
